# Initial kernel scaffold; baseline (speedup 1.0000x reference)
import jax
import jax.numpy as jnp
from jax.experimental import pallas as pl


def kernel(seq, token_table, pos_table):
    raise NotImplementedError("write your pallas kernel here")


# speedup vs baseline: 2.6320x; 2.6320x over previous
import functools

import jax
import jax.numpy as jnp
from jax import lax
from jax.experimental import pallas as pl
from jax.experimental.pallas import tpu as pltpu
from jax.experimental.pallas import tpu_sc as plsc

VOCAB = 100000
MAX_LEN = 200
EMB = 128
B = 1024
T = 200

NC = 2
NS = 16
NW = NC * NS

CHUNK = 100
ROWS = (B * T) // CHUNK
RPW = ROWS // NW

TOK_BLK = 2000
GRID = VOCAB // TOK_BLK


def _prep_body(tok_ref, pos_ref, m_ref, qpos_ref, acc_ref):
    i = pl.program_id(0)

    @pl.when(i == 0)
    def _():
        acc_ref[0, 0] = 0.0

    acc_ref[0, 0] += jnp.sum(tok_ref[...])
    m_tok = acc_ref[0, 0] / float(VOCAB * EMB)
    m_ref[...] = jnp.full((8, 128), m_tok, jnp.float32)
    pos = pos_ref[...]
    m_pos = jnp.sum(pos) / float(MAX_LEN * EMB)
    qpos_ref[...] = (pos > m_pos).astype(jnp.float32)


def _prep(token_table, pos_table):
    return pl.pallas_call(
        _prep_body,
        grid=(GRID,),
        in_specs=[
            pl.BlockSpec((TOK_BLK, EMB), lambda i: (i, 0)),
            pl.BlockSpec((MAX_LEN, EMB), lambda i: (0, 0)),
        ],
        out_specs=[
            pl.BlockSpec((8, 128), lambda i: (0, 0)),
            pl.BlockSpec((MAX_LEN, EMB), lambda i: (0, 0)),
        ],
        out_shape=[
            jax.ShapeDtypeStruct((8, 128), jnp.float32),
            jax.ShapeDtypeStruct((MAX_LEN, EMB), jnp.float32),
        ],
        scratch_shapes=[pltpu.SMEM((1, 1), jnp.float32)],
    )(token_table, pos_table)


_mesh = plsc.VectorSubcoreMesh(
    core_axis_name="c", subcore_axis_name="s", num_cores=NC, num_subcores=NS
)


@functools.partial(
    pl.kernel,
    out_type=jax.ShapeDtypeStruct((ROWS, CHUNK, EMB), jnp.float32),
    mesh=_mesh,
    scratch_types=[
        pltpu.VMEM((CHUNK,), jnp.int32),
        pltpu.VMEM((CHUNK, EMB), jnp.float32),
        pltpu.VMEM((MAX_LEN, EMB), jnp.float32),
        pltpu.VMEM((16,), jnp.float32),
        pltpu.SemaphoreType.DMA,
    ],
)
def _sc_lookup(seq_hbm, tok_hbm, m_hbm, qpos_hbm, out_hbm,
               idx_v, g_v, qpos_v, m_v, gsem):
    wid = lax.axis_index("s") * NC + lax.axis_index("c")
    base = wid * RPW

    pltpu.sync_copy(qpos_hbm, qpos_v)
    pltpu.sync_copy(m_hbm, m_v)
    vm = m_v[...]

    def do_row(i, carry):
        row = base + i
        pltpu.sync_copy(seq_hbm.at[row], idx_v)
        pltpu.async_copy(tok_hbm.at[idx_v], g_v, gsem).wait()
        off = lax.rem(row, 2) * CHUNK

        def body_r(r, c):
            tr = off + r
            for j in range(EMB // 16):
                sl = pl.ds(j * 16, 16)
                v = g_v[r, sl]
                qp = qpos_v[tr, sl]
                g_v[r, sl] = jnp.where(v > vm, 1.0, 0.0) + qp
            return c

        lax.fori_loop(0, CHUNK, body_r, 0)
        pltpu.sync_copy(g_v, out_hbm.at[row])
        return carry

    lax.fori_loop(0, RPW, do_row, 0)


def kernel(seq, token_table, pos_table):
    m8, qpos = _prep(token_table, pos_table)
    mvec = m8[0, :16]
    seq2 = seq.reshape(ROWS, CHUNK).astype(jnp.int32)
    out = _sc_lookup(seq2, token_table, mvec, qpos)
    return out.reshape(B, T, EMB)


# speedup vs baseline: 5.9755x; 2.2703x over previous
import functools

import jax
import jax.numpy as jnp
from jax import lax
from jax.experimental import pallas as pl
from jax.experimental.pallas import tpu as pltpu
from jax.experimental.pallas import tpu_sc as plsc

VOCAB = 100000
MAX_LEN = 200
EMB = 128
B = 1024
T = 200

NC = 2
NS = 16
NW = NC * NS

CHUNK = 100
ROWS = (B * T) // CHUNK
RPW = ROWS // NW

TOK_BLK = 2000
GRID = VOCAB // TOK_BLK


def _prep_body(tok_ref, pos_ref, m_ref, qpos_ref, acc_ref):
    i = pl.program_id(0)

    @pl.when(i == 0)
    def _():
        acc_ref[0, 0] = 0.0

    acc_ref[0, 0] += jnp.sum(tok_ref[...])
    m_tok = acc_ref[0, 0] / float(VOCAB * EMB)
    m_ref[...] = jnp.full((8, 128), m_tok, jnp.float32)
    pos = pos_ref[...]
    m_pos = jnp.sum(pos) / float(MAX_LEN * EMB)
    qpos_ref[...] = (pos > m_pos).astype(jnp.float32)


def _prep(token_table, pos_table):
    return pl.pallas_call(
        _prep_body,
        grid=(GRID,),
        in_specs=[
            pl.BlockSpec((TOK_BLK, EMB), lambda i: (i, 0)),
            pl.BlockSpec((MAX_LEN, EMB), lambda i: (0, 0)),
        ],
        out_specs=[
            pl.BlockSpec((8, 128), lambda i: (0, 0)),
            pl.BlockSpec((MAX_LEN, EMB), lambda i: (0, 0)),
        ],
        out_shape=[
            jax.ShapeDtypeStruct((8, 128), jnp.float32),
            jax.ShapeDtypeStruct((MAX_LEN, EMB), jnp.float32),
        ],
        scratch_shapes=[pltpu.SMEM((1, 1), jnp.float32)],
    )(token_table, pos_table)


_mesh = plsc.VectorSubcoreMesh(
    core_axis_name="c", subcore_axis_name="s", num_cores=NC, num_subcores=NS
)


@functools.partial(
    pl.kernel,
    out_type=jax.ShapeDtypeStruct((ROWS, CHUNK, EMB), jnp.float32),
    mesh=_mesh,
    scratch_types=[
        pltpu.VMEM((RPW, CHUNK), jnp.int32),
        pltpu.VMEM((CHUNK, EMB), jnp.float32),
        pltpu.VMEM((CHUNK, EMB), jnp.float32),
        pltpu.VMEM((CHUNK, EMB), jnp.float32),
        pltpu.VMEM((CHUNK, EMB), jnp.float32),
        pltpu.VMEM((MAX_LEN, EMB), jnp.float32),
        pltpu.VMEM((16,), jnp.float32),
        pltpu.SemaphoreType.DMA,
        pltpu.SemaphoreType.DMA,
        pltpu.SemaphoreType.DMA,
        pltpu.SemaphoreType.DMA,
    ],
)
def _sc_lookup(seq_hbm, tok_hbm, m_hbm, qpos_hbm, out_hbm,
               idx_all, g0, g1, o0, o1, qpos_v, m_v,
               gsem0, gsem1, osem0, osem1):
    wid = lax.axis_index("s") * NC + lax.axis_index("c")
    base = wid * RPW

    pltpu.sync_copy(seq_hbm.at[pl.ds(base, RPW)], idx_all)
    pltpu.sync_copy(qpos_hbm, qpos_v)
    pltpu.sync_copy(m_hbm, m_v)
    vm = m_v[...]

    def gstart(i, g, gsem):
        pltpu.async_copy(tok_hbm.at[idx_all.at[i]], g, gsem)

    def gwait(g, gsem):
        pltpu.make_async_copy(tok_hbm.at[idx_all.at[0]], g, gsem).wait()

    def ostart(i, o, osem):
        pltpu.async_copy(o, out_hbm.at[base + i], osem)

    def owait(o, osem):
        pltpu.make_async_copy(o, out_hbm.at[base], osem).wait()

    def compute(i, g, o):
        off = lax.rem(i, 2) * CHUNK

        def body_r(r, c):
            tr = off + r
            for j in range(EMB // 16):
                sl = pl.ds(j * 16, 16)
                v = g[r, sl]
                qp = qpos_v[tr, sl]
                o[r, sl] = jnp.where(v > vm, 1.0, 0.0) + qp
            return c

        lax.fori_loop(0, CHUNK, body_r, 0)

    def slot(i, g, o, gsem, osem, nxt, first, last):
        gwait(g, gsem)
        if not first:
            owait(o, osem)
        compute(i, g, o)
        ostart(i, o, osem)
        if not last:
            gstart(nxt, g, gsem)

    gstart(0, g0, gsem0)
    gstart(1, g1, gsem1)
    slot(0, g0, o0, gsem0, osem0, 2, True, False)
    slot(1, g1, o1, gsem1, osem1, 3, True, False)

    def pair(k, carry):
        e = 2 * k
        slot(e, g0, o0, gsem0, osem0, e + 2, False, False)
        slot(e + 1, g1, o1, gsem1, osem1, e + 3, False, False)
        return carry

    lax.fori_loop(1, RPW // 2 - 1, pair, 0)
    slot(RPW - 2, g0, o0, gsem0, osem0, 0, False, True)
    slot(RPW - 1, g1, o1, gsem1, osem1, 0, False, True)
    owait(o0, osem0)
    owait(o1, osem1)


def kernel(seq, token_table, pos_table):
    m8, qpos = _prep(token_table, pos_table)
    mvec = m8[0, :16]
    seq2 = seq.reshape(ROWS, CHUNK).astype(jnp.int32)
    out = _sc_lookup(seq2, token_table, mvec, qpos)
    return out.reshape(B, T, EMB)
